# baseline (speedup 1.0000x reference)
import functools

import jax
import jax.numpy as jnp
from jax import lax
from jax.experimental import pallas as pl
from jax.experimental.pallas import tpu as pltpu
from jax.experimental.pallas import tpu_sc as plsc

_RADIUS2 = 0.1 * 0.1
_NSAMPLE = 32
_L = 16
_K = 4
_CB = 64


def _ball_query_body(nb, pb, n, p_total, xyz_t_hbm, nxyz_t_hbm, idx_hbm,
                     gxyz_hbm, x_ref, y_ref, z_ref, cx_ref, cy_ref, cz_ref,
                     stage_ref, gstage_ref, tmp0_ref, tmp1_ref, tmp2_ref,
                     tmp3_ref):
    tmp_refs = (tmp0_ref, tmp1_ref, tmp2_ref, tmp3_ref)
    wid = lax.axis_index("s") * 2 + lax.axis_index("c")
    b = wid // nb
    cb = wid % nb
    pltpu.sync_copy(xyz_t_hbm.at[pl.ds((b * 3 + 0) * n, n)], x_ref)
    pltpu.sync_copy(xyz_t_hbm.at[pl.ds((b * 3 + 1) * n, n)], y_ref)
    pltpu.sync_copy(xyz_t_hbm.at[pl.ds((b * 3 + 2) * n, n)], z_ref)
    cstart = cb * pb
    pltpu.sync_copy(nxyz_t_hbm.at[pl.ds((b * 3 + 0) * p_total + cstart, pb)],
                    cx_ref)
    pltpu.sync_copy(nxyz_t_hbm.at[pl.ds((b * 3 + 1) * p_total + cstart, pb)],
                    cy_ref)
    pltpu.sync_copy(nxyz_t_hbm.at[pl.ds((b * 3 + 2) * p_total + cstart, pb)],
                    cz_ref)
    iota = lax.iota(jnp.int32, _L)
    zeros16 = jnp.zeros((_L,), jnp.int32)
    ones16 = jnp.ones((_L,), jnp.int32)
    nblk = n // (_L * _CB)

    def per_group(g, carry):
        p0 = g * _K
        cxs, cys, czs = [], [], []
        for k in range(_K):
            pv = zeros16 + (p0 + k)
            cxs.append(plsc.load_gather(cx_ref, [pv]))
            cys.append(plsc.load_gather(cy_ref, [pv]))
            czs.append(plsc.load_gather(cz_ref, [pv]))
            tmp_refs[k][pl.ds(0, _L)] = zeros16

        def chunk(j, cnts):
            base = j * _L
            px = x_ref[pl.ds(base, _L)]
            py = y_ref[pl.ds(base, _L)]
            pz = z_ref[pl.ds(base, _L)]
            iv = iota + base
            out = []
            for k in range(_K):
                dx = cxs[k] - px
                dy = cys[k] - py
                dz = czs[k] - pz
                d2 = (dx * dx + dy * dy) + dz * dz
                m = d2 < _RADIUS2
                incl = plsc.cumsum(ones16, mask=m)
                pos = jnp.minimum(cnts[k] + incl, 47)
                plsc.store_scatter(tmp_refs[k], [pos], iv, mask=m)
                out.append(cnts[k] + plsc.all_reduce_population_count(m))
            return tuple(out)

        def blk_cond(state):
            blk = state[0]
            cnts = state[1:]
            cmin = jnp.minimum(jnp.minimum(cnts[0], cnts[1]),
                               jnp.minimum(cnts[2], cnts[3]))
            return jnp.logical_and(blk < nblk, jnp.min(cmin) < _NSAMPLE - 1)

        def blk_body(state):
            blk = state[0]
            cnts = state[1:]
            cnts = plsc.parallel_loop(blk * _CB, (blk + 1) * _CB, unroll=8,
                                      carry=cnts)(chunk)
            return (blk + 1,) + cnts

        state = (jnp.int32(0),) + tuple(zeros16 - 1 for _ in range(_K))
        state = lax.while_loop(blk_cond, blk_body, state)
        cnts_fin = state[1:]

        for k in range(_K):
            cnt = cnts_fin[k] + 1
            v0 = tmp_refs[k][pl.ds(0, _L)]
            first = zeros16 + jnp.min(
                jnp.where(iota == 0, v0, jnp.int32(2**30)))
            for h in range(_NSAMPLE // _L):
                lane = iota + (_L * h)
                v = tmp_refs[k][pl.ds(_L * h, _L)]
                ov = jnp.where(lane < cnt, v, first)
                stage_ref[pl.ds((p0 + k) * _NSAMPLE + _L * h, _L)] = ov
                goff = (p0 + k) * _NSAMPLE + _L * h
                gstage_ref[pl.ds(goff, _L)] = (
                    plsc.load_gather(x_ref, [ov]) - cxs[k])
                gstage_ref[pl.ds(pb * _NSAMPLE + goff, _L)] = (
                    plsc.load_gather(y_ref, [ov]) - cys[k])
                gstage_ref[pl.ds(2 * pb * _NSAMPLE + goff, _L)] = (
                    plsc.load_gather(z_ref, [ov]) - czs[k])
        return carry

    lax.fori_loop(0, pb // _K, per_group, 0)
    pltpu.sync_copy(
        stage_ref,
        idx_hbm.at[pl.ds((b * p_total + cstart) * _NSAMPLE, pb * _NSAMPLE)])
    for ch in range(3):
        pltpu.sync_copy(
            gstage_ref.at[pl.ds(ch * pb * _NSAMPLE, pb * _NSAMPLE)],
            gxyz_hbm.at[pl.ds(((b * 3 + ch) * p_total + cstart) * _NSAMPLE,
                              pb * _NSAMPLE)])


def _group_body(nb, nrows, n, p_total, feat_hbm, idx_hbm, gxyz_hbm, out_hbm,
                row_ref, idx_ref, ostage_ref):
    wid = lax.axis_index("s") * 2 + lax.axis_index("c")
    b = wid // nb
    l = wid % nb
    nfeat = nrows - 3
    pchunk = p_total * _NSAMPLE
    pltpu.sync_copy(idx_hbm.at[pl.ds(b * pchunk, pchunk)], idx_ref)

    @pl.when(l < 3)
    def _():
        pltpu.sync_copy(gxyz_hbm.at[pl.ds((b * 3 + l) * pchunk, pchunk)],
                        ostage_ref)
        pltpu.sync_copy(ostage_ref, out_hbm.at[pl.ds((b * nrows + l) * pchunk,
                                                     pchunk)])

    def per_task(t, carry):
        c = l + nb * t
        pltpu.sync_copy(feat_hbm.at[pl.ds((b * nfeat + c) * n, n)], row_ref)

        @plsc.parallel_loop(0, p_total, unroll=4)
        def per_p(p):
            for h in range(_NSAMPLE // _L):
                off = p * _NSAMPLE + _L * h
                iv = idx_ref[pl.ds(off, _L)]
                ostage_ref[pl.ds(off, _L)] = plsc.load_gather(row_ref, [iv])

        pltpu.sync_copy(ostage_ref,
                        out_hbm.at[pl.ds((b * nrows + 3 + c) * pchunk,
                                         pchunk)])
        return carry

    lax.fori_loop(0, nfeat // nb, per_task, 0)


def kernel(xyz, new_xyz, features):
    B, N, _ = xyz.shape
    P = new_xyz.shape[1]
    C = features.shape[1]
    nrows = C + 3
    nb = 32 // B
    pb = P // nb
    mesh = plsc.VectorSubcoreMesh(core_axis_name="c", subcore_axis_name="s",
                                  num_cores=2, num_subcores=16)

    xyz_t = jnp.transpose(xyz, (0, 2, 1)).reshape(-1)
    nxyz_t = jnp.transpose(new_xyz, (0, 2, 1)).reshape(-1)
    feat_flat = features.reshape(-1)

    ballq = pl.kernel(
        functools.partial(_ball_query_body, nb, pb, N, P),
        out_type=(
            jax.ShapeDtypeStruct((B * P * _NSAMPLE,), jnp.int32),
            jax.ShapeDtypeStruct((B * 3 * P * _NSAMPLE,), jnp.float32),
        ),
        mesh=mesh,
        compiler_params=pltpu.CompilerParams(needs_layout_passes=False),
        scratch_types=[
            pltpu.VMEM((N,), jnp.float32),
            pltpu.VMEM((N,), jnp.float32),
            pltpu.VMEM((N,), jnp.float32),
            pltpu.VMEM((pb,), jnp.float32),
            pltpu.VMEM((pb,), jnp.float32),
            pltpu.VMEM((pb,), jnp.float32),
            pltpu.VMEM((pb * _NSAMPLE,), jnp.int32),
            pltpu.VMEM((3 * pb * _NSAMPLE,), jnp.float32),
            pltpu.VMEM((48,), jnp.int32),
            pltpu.VMEM((48,), jnp.int32),
            pltpu.VMEM((48,), jnp.int32),
            pltpu.VMEM((48,), jnp.int32),
        ],
    )
    idx, gxyz = ballq(xyz_t, nxyz_t)

    group = pl.kernel(
        functools.partial(_group_body, nb, nrows, N, P),
        out_type=jax.ShapeDtypeStruct((B * nrows * P * _NSAMPLE,),
                                      jnp.float32),
        mesh=mesh,
        compiler_params=pltpu.CompilerParams(needs_layout_passes=False),
        scratch_types=[
            pltpu.VMEM((N,), jnp.float32),
            pltpu.VMEM((P * _NSAMPLE,), jnp.int32),
            pltpu.VMEM((P * _NSAMPLE,), jnp.float32),
        ],
    )
    out = group(feat_flat, idx, gxyz)
    return out.reshape(B, nrows, P, _NSAMPLE)


# speedup vs baseline: 1.9964x; 1.9964x over previous
import functools

import jax
import jax.numpy as jnp
from jax import lax
from jax.experimental import pallas as pl
from jax.experimental.pallas import tpu as pltpu
from jax.experimental.pallas import tpu_sc as plsc

_RADIUS2 = 0.1 * 0.1
_NSAMPLE = 32
_L = 16
_K = 4
_CB = 64


def _ball_query_body(nb, pb, n, p_total, xyz_t_hbm, nxyz_t_hbm, idx_hbm,
                     gxyz_hbm, x_ref, y_ref, z_ref, cx_ref, cy_ref, cz_ref,
                     stage_ref, gstage_ref, tmp0_ref, tmp1_ref, tmp2_ref,
                     tmp3_ref):
    tmp_refs = (tmp0_ref, tmp1_ref, tmp2_ref, tmp3_ref)
    wid = lax.axis_index("s") * 2 + lax.axis_index("c")
    b = wid // nb
    cb = wid % nb
    pltpu.sync_copy(xyz_t_hbm.at[pl.ds((b * 3 + 0) * n, n)], x_ref)
    pltpu.sync_copy(xyz_t_hbm.at[pl.ds((b * 3 + 1) * n, n)], y_ref)
    pltpu.sync_copy(xyz_t_hbm.at[pl.ds((b * 3 + 2) * n, n)], z_ref)
    cstart = cb * pb
    pltpu.sync_copy(nxyz_t_hbm.at[pl.ds((b * 3 + 0) * p_total + cstart, pb)],
                    cx_ref)
    pltpu.sync_copy(nxyz_t_hbm.at[pl.ds((b * 3 + 1) * p_total + cstart, pb)],
                    cy_ref)
    pltpu.sync_copy(nxyz_t_hbm.at[pl.ds((b * 3 + 2) * p_total + cstart, pb)],
                    cz_ref)
    iota = lax.iota(jnp.int32, _L)
    zeros16 = jnp.zeros((_L,), jnp.int32)
    ones16 = jnp.ones((_L,), jnp.int32)
    nblk = n // (_L * _CB)

    def per_group(g, carry):
        p0 = g * _K
        cxs, cys, czs = [], [], []
        for k in range(_K):
            pv = zeros16 + (p0 + k)
            cxs.append(plsc.load_gather(cx_ref, [pv]))
            cys.append(plsc.load_gather(cy_ref, [pv]))
            czs.append(plsc.load_gather(cz_ref, [pv]))
            tmp_refs[k][pl.ds(0, _L)] = zeros16

        def chunk(j, cnts):
            base = j * _L
            px = x_ref[pl.ds(base, _L)]
            py = y_ref[pl.ds(base, _L)]
            pz = z_ref[pl.ds(base, _L)]
            iv = iota + base
            out = []
            for k in range(_K):
                dx = cxs[k] - px
                dy = cys[k] - py
                dz = czs[k] - pz
                d2 = (dx * dx + dy * dy) + dz * dz
                m = d2 < _RADIUS2
                incl = plsc.cumsum(ones16, mask=m)
                pos = jnp.minimum(cnts[k] + incl, 47)
                plsc.store_scatter(tmp_refs[k], [pos], iv, mask=m)
                out.append(cnts[k] + plsc.all_reduce_population_count(m))
            return tuple(out)

        def blk_cond(state):
            blk = state[0]
            cnts = state[1:]
            cmin = jnp.minimum(jnp.minimum(cnts[0], cnts[1]),
                               jnp.minimum(cnts[2], cnts[3]))
            return jnp.logical_and(blk < nblk, jnp.min(cmin) < _NSAMPLE - 1)

        def blk_body(state):
            blk = state[0]
            cnts = state[1:]
            cnts = plsc.parallel_loop(blk * _CB, (blk + 1) * _CB, unroll=4,
                                      carry=cnts)(chunk)
            return (blk + 1,) + cnts

        state = (jnp.int32(0),) + tuple(zeros16 - 1 for _ in range(_K))
        state = lax.while_loop(blk_cond, blk_body, state)
        cnts_fin = state[1:]

        for k in range(_K):
            cnt = cnts_fin[k] + 1
            v0 = tmp_refs[k][pl.ds(0, _L)]
            first = zeros16 + jnp.min(
                jnp.where(iota == 0, v0, jnp.int32(2**30)))
            for h in range(_NSAMPLE // _L):
                lane = iota + (_L * h)
                v = tmp_refs[k][pl.ds(_L * h, _L)]
                ov = jnp.where(lane < cnt, v, first)
                stage_ref[pl.ds((p0 + k) * _NSAMPLE + _L * h, _L)] = ov
                goff = (p0 + k) * _NSAMPLE + _L * h
                gstage_ref[pl.ds(goff, _L)] = (
                    plsc.load_gather(x_ref, [ov]) - cxs[k])
                gstage_ref[pl.ds(pb * _NSAMPLE + goff, _L)] = (
                    plsc.load_gather(y_ref, [ov]) - cys[k])
                gstage_ref[pl.ds(2 * pb * _NSAMPLE + goff, _L)] = (
                    plsc.load_gather(z_ref, [ov]) - czs[k])
        return carry

    lax.fori_loop(0, pb // _K, per_group, 0)
    pltpu.sync_copy(
        stage_ref,
        idx_hbm.at[pl.ds((b * p_total + cstart) * _NSAMPLE, pb * _NSAMPLE)])
    for ch in range(3):
        pltpu.sync_copy(
            gstage_ref.at[pl.ds(ch * pb * _NSAMPLE, pb * _NSAMPLE)],
            gxyz_hbm.at[pl.ds(((b * 3 + ch) * p_total + cstart) * _NSAMPLE,
                              pb * _NSAMPLE)])


def _group_body(nb, nrows, n, p_total, feat_hbm, idx_hbm, gxyz_hbm, out_hbm,
                row_ref, idx_ref, ostage_ref):
    wid = lax.axis_index("s") * 2 + lax.axis_index("c")
    b = wid // nb
    l = wid % nb
    nfeat = nrows - 3
    pchunk = p_total * _NSAMPLE
    pltpu.sync_copy(idx_hbm.at[pl.ds(b * pchunk, pchunk)], idx_ref)

    @pl.when(l < 3)
    def _():
        pltpu.sync_copy(gxyz_hbm.at[pl.ds((b * 3 + l) * pchunk, pchunk)],
                        ostage_ref)
        pltpu.sync_copy(ostage_ref, out_hbm.at[pl.ds((b * nrows + l) * pchunk,
                                                     pchunk)])

    def per_task(t, carry):
        c = l + nb * t
        pltpu.sync_copy(feat_hbm.at[pl.ds((b * nfeat + c) * n, n)], row_ref)

        @plsc.parallel_loop(0, p_total, unroll=4)
        def per_p(p):
            for h in range(_NSAMPLE // _L):
                off = p * _NSAMPLE + _L * h
                iv = idx_ref[pl.ds(off, _L)]
                ostage_ref[pl.ds(off, _L)] = plsc.load_gather(row_ref, [iv])

        pltpu.sync_copy(ostage_ref,
                        out_hbm.at[pl.ds((b * nrows + 3 + c) * pchunk,
                                         pchunk)])
        return carry

    lax.fori_loop(0, nfeat // nb, per_task, 0)


def kernel(xyz, new_xyz, features):
    B, N, _ = xyz.shape
    P = new_xyz.shape[1]
    C = features.shape[1]
    nrows = C + 3
    nb = 32 // B
    pb = P // nb
    mesh = plsc.VectorSubcoreMesh(core_axis_name="c", subcore_axis_name="s",
                                  num_cores=2, num_subcores=16)

    xyz_t = jnp.transpose(xyz, (0, 2, 1)).reshape(-1)
    nxyz_t = jnp.transpose(new_xyz, (0, 2, 1)).reshape(-1)
    feat_flat = features.reshape(-1)

    ballq = pl.kernel(
        functools.partial(_ball_query_body, nb, pb, N, P),
        out_type=(
            jax.ShapeDtypeStruct((B * P * _NSAMPLE,), jnp.int32),
            jax.ShapeDtypeStruct((B * 3 * P * _NSAMPLE,), jnp.float32),
        ),
        mesh=mesh,
        compiler_params=pltpu.CompilerParams(needs_layout_passes=False),
        scratch_types=[
            pltpu.VMEM((N,), jnp.float32),
            pltpu.VMEM((N,), jnp.float32),
            pltpu.VMEM((N,), jnp.float32),
            pltpu.VMEM((pb,), jnp.float32),
            pltpu.VMEM((pb,), jnp.float32),
            pltpu.VMEM((pb,), jnp.float32),
            pltpu.VMEM((pb * _NSAMPLE,), jnp.int32),
            pltpu.VMEM((3 * pb * _NSAMPLE,), jnp.float32),
            pltpu.VMEM((48,), jnp.int32),
            pltpu.VMEM((48,), jnp.int32),
            pltpu.VMEM((48,), jnp.int32),
            pltpu.VMEM((48,), jnp.int32),
        ],
    )
    idx, gxyz = ballq(xyz_t, nxyz_t)

    group = pl.kernel(
        functools.partial(_group_body, nb, nrows, N, P),
        out_type=jax.ShapeDtypeStruct((B * nrows * P * _NSAMPLE,),
                                      jnp.float32),
        mesh=mesh,
        compiler_params=pltpu.CompilerParams(needs_layout_passes=False),
        scratch_types=[
            pltpu.VMEM((N,), jnp.float32),
            pltpu.VMEM((P * _NSAMPLE,), jnp.int32),
            pltpu.VMEM((P * _NSAMPLE,), jnp.float32),
        ],
    )
    out = group(feat_flat, idx, gxyz)
    return out.reshape(B, nrows, P, _NSAMPLE)


# speedup vs baseline: 2.2919x; 1.1480x over previous
import functools

import jax
import jax.numpy as jnp
from jax import lax
from jax.experimental import pallas as pl
from jax.experimental.pallas import tpu as pltpu
from jax.experimental.pallas import tpu_sc as plsc

_RADIUS2 = 0.1 * 0.1
_NSAMPLE = 32
_L = 16
_K = 4
_CB = 64


def _ball_query_body(nb, pb, n, p_total, xyz_t_hbm, nxyz_t_hbm, idx_hbm,
                     gxyz_hbm, x_ref, y_ref, z_ref, cx_ref, cy_ref, cz_ref,
                     stage_ref, gstage_ref, tmp0_ref, tmp1_ref, tmp2_ref,
                     tmp3_ref):
    tmp_refs = (tmp0_ref, tmp1_ref, tmp2_ref, tmp3_ref)
    wid = lax.axis_index("s") * 2 + lax.axis_index("c")
    b = wid // nb
    cb = wid % nb
    pltpu.sync_copy(xyz_t_hbm.at[pl.ds((b * 3 + 0) * n, n)], x_ref)
    pltpu.sync_copy(xyz_t_hbm.at[pl.ds((b * 3 + 1) * n, n)], y_ref)
    pltpu.sync_copy(xyz_t_hbm.at[pl.ds((b * 3 + 2) * n, n)], z_ref)
    cstart = cb * pb
    pltpu.sync_copy(nxyz_t_hbm.at[pl.ds((b * 3 + 0) * p_total + cstart, pb)],
                    cx_ref)
    pltpu.sync_copy(nxyz_t_hbm.at[pl.ds((b * 3 + 1) * p_total + cstart, pb)],
                    cy_ref)
    pltpu.sync_copy(nxyz_t_hbm.at[pl.ds((b * 3 + 2) * p_total + cstart, pb)],
                    cz_ref)
    iota = lax.iota(jnp.int32, _L)
    zeros16 = jnp.zeros((_L,), jnp.int32)
    ones16 = jnp.ones((_L,), jnp.int32)
    nblk = n // (_L * _CB)

    def per_group(g, carry):
        p0 = g * _K
        cxs, cys, czs = [], [], []
        for k in range(_K):
            pv = zeros16 + (p0 + k)
            cxs.append(plsc.load_gather(cx_ref, [pv]))
            cys.append(plsc.load_gather(cy_ref, [pv]))
            czs.append(plsc.load_gather(cz_ref, [pv]))
            tmp_refs[k][pl.ds(0, _L)] = zeros16

        def chunk(j, cnts):
            base = j * _L
            px = x_ref[pl.ds(base, _L)]
            py = y_ref[pl.ds(base, _L)]
            pz = z_ref[pl.ds(base, _L)]
            iv = iota + base
            out = []
            for k in range(_K):
                dx = cxs[k] - px
                dy = cys[k] - py
                dz = czs[k] - pz
                d2 = (dx * dx + dy * dy) + dz * dz
                m = d2 < _RADIUS2
                incl = plsc.cumsum(m.astype(jnp.int32))
                pos = jnp.minimum(jnp.maximum(cnts[k] + incl, 0), 47)
                plsc.store_scatter(tmp_refs[k], [pos], iv, mask=m)
                out.append(cnts[k] + plsc.all_reduce_population_count(m))
            return tuple(out)

        def blk_cond(state):
            blk = state[0]
            cnts = state[1:]
            cmin = jnp.minimum(jnp.minimum(cnts[0], cnts[1]),
                               jnp.minimum(cnts[2], cnts[3]))
            return jnp.logical_and(blk < nblk, jnp.min(cmin) < _NSAMPLE - 1)

        def blk_body(state):
            blk = state[0]
            cnts = state[1:]
            cnts = plsc.parallel_loop(blk * _CB, (blk + 1) * _CB, unroll=4,
                                      carry=cnts)(chunk)
            return (blk + 1,) + cnts

        state = (jnp.int32(0),) + tuple(zeros16 - 1 for _ in range(_K))
        state = lax.while_loop(blk_cond, blk_body, state)
        cnts_fin = state[1:]

        for k in range(_K):
            cnt = cnts_fin[k] + 1
            v0 = tmp_refs[k][pl.ds(0, _L)]
            first = zeros16 + jnp.min(
                jnp.where(iota == 0, v0, jnp.int32(2**30)))
            for h in range(_NSAMPLE // _L):
                lane = iota + (_L * h)
                v = tmp_refs[k][pl.ds(_L * h, _L)]
                ov = jnp.where(lane < cnt, v, first)
                stage_ref[pl.ds((p0 + k) * _NSAMPLE + _L * h, _L)] = ov
                goff = (p0 + k) * _NSAMPLE + _L * h
                gstage_ref[pl.ds(goff, _L)] = (
                    plsc.load_gather(x_ref, [ov]) - cxs[k])
                gstage_ref[pl.ds(pb * _NSAMPLE + goff, _L)] = (
                    plsc.load_gather(y_ref, [ov]) - cys[k])
                gstage_ref[pl.ds(2 * pb * _NSAMPLE + goff, _L)] = (
                    plsc.load_gather(z_ref, [ov]) - czs[k])
        return carry

    lax.fori_loop(0, pb // _K, per_group, 0)
    pltpu.sync_copy(
        stage_ref,
        idx_hbm.at[pl.ds((b * p_total + cstart) * _NSAMPLE, pb * _NSAMPLE)])
    for ch in range(3):
        pltpu.sync_copy(
            gstage_ref.at[pl.ds(ch * pb * _NSAMPLE, pb * _NSAMPLE)],
            gxyz_hbm.at[pl.ds(((b * 3 + ch) * p_total + cstart) * _NSAMPLE,
                              pb * _NSAMPLE)])


def _group_body(nb, nrows, n, p_total, feat_hbm, idx_hbm, gxyz_hbm, out_hbm,
                row_ref, idx_ref, ostage_ref):
    wid = lax.axis_index("s") * 2 + lax.axis_index("c")
    b = wid // nb
    l = wid % nb
    nfeat = nrows - 3
    pchunk = p_total * _NSAMPLE
    pltpu.sync_copy(idx_hbm.at[pl.ds(b * pchunk, pchunk)], idx_ref)

    @pl.when(l < 3)
    def _():
        pltpu.sync_copy(gxyz_hbm.at[pl.ds((b * 3 + l) * pchunk, pchunk)],
                        ostage_ref)
        pltpu.sync_copy(ostage_ref, out_hbm.at[pl.ds((b * nrows + l) * pchunk,
                                                     pchunk)])

    def per_task(t, carry):
        c = l + nb * t
        pltpu.sync_copy(feat_hbm.at[pl.ds((b * nfeat + c) * n, n)], row_ref)

        @plsc.parallel_loop(0, p_total, unroll=4)
        def per_p(p):
            for h in range(_NSAMPLE // _L):
                off = p * _NSAMPLE + _L * h
                iv = idx_ref[pl.ds(off, _L)]
                ostage_ref[pl.ds(off, _L)] = plsc.load_gather(row_ref, [iv])

        pltpu.sync_copy(ostage_ref,
                        out_hbm.at[pl.ds((b * nrows + 3 + c) * pchunk,
                                         pchunk)])
        return carry

    lax.fori_loop(0, nfeat // nb, per_task, 0)


def kernel(xyz, new_xyz, features):
    B, N, _ = xyz.shape
    P = new_xyz.shape[1]
    C = features.shape[1]
    nrows = C + 3
    nb = 32 // B
    pb = P // nb
    mesh = plsc.VectorSubcoreMesh(core_axis_name="c", subcore_axis_name="s",
                                  num_cores=2, num_subcores=16)

    xyz_t = jnp.transpose(xyz, (0, 2, 1)).reshape(-1)
    nxyz_t = jnp.transpose(new_xyz, (0, 2, 1)).reshape(-1)
    feat_flat = features.reshape(-1)

    ballq = pl.kernel(
        functools.partial(_ball_query_body, nb, pb, N, P),
        out_type=(
            jax.ShapeDtypeStruct((B * P * _NSAMPLE,), jnp.int32),
            jax.ShapeDtypeStruct((B * 3 * P * _NSAMPLE,), jnp.float32),
        ),
        mesh=mesh,
        compiler_params=pltpu.CompilerParams(needs_layout_passes=False),
        scratch_types=[
            pltpu.VMEM((N,), jnp.float32),
            pltpu.VMEM((N,), jnp.float32),
            pltpu.VMEM((N,), jnp.float32),
            pltpu.VMEM((pb,), jnp.float32),
            pltpu.VMEM((pb,), jnp.float32),
            pltpu.VMEM((pb,), jnp.float32),
            pltpu.VMEM((pb * _NSAMPLE,), jnp.int32),
            pltpu.VMEM((3 * pb * _NSAMPLE,), jnp.float32),
            pltpu.VMEM((48,), jnp.int32),
            pltpu.VMEM((48,), jnp.int32),
            pltpu.VMEM((48,), jnp.int32),
            pltpu.VMEM((48,), jnp.int32),
        ],
    )
    idx, gxyz = ballq(xyz_t, nxyz_t)

    group = pl.kernel(
        functools.partial(_group_body, nb, nrows, N, P),
        out_type=jax.ShapeDtypeStruct((B * nrows * P * _NSAMPLE,),
                                      jnp.float32),
        mesh=mesh,
        compiler_params=pltpu.CompilerParams(needs_layout_passes=False),
        scratch_types=[
            pltpu.VMEM((N,), jnp.float32),
            pltpu.VMEM((P * _NSAMPLE,), jnp.int32),
            pltpu.VMEM((P * _NSAMPLE,), jnp.float32),
        ],
    )
    out = group(feat_flat, idx, gxyz)
    return out.reshape(B, nrows, P, _NSAMPLE)


# speedup vs baseline: 2.3342x; 1.0185x over previous
import functools

import jax
import jax.numpy as jnp
from jax import lax
from jax.experimental import pallas as pl
from jax.experimental.pallas import tpu as pltpu
from jax.experimental.pallas import tpu_sc as plsc

_RADIUS2 = 0.1 * 0.1
_NSAMPLE = 32
_L = 16
_K = 4
_CB = 64
_Q = 4


def _fused_body(nb, pb, n, p_total, nrows, xyz_t_hbm, nxyz_t_hbm, feat_hbm,
                out_hbm, x_ref, y_ref, z_ref, cx_ref, cy_ref, cz_ref,
                stage_ref, gstage_ref, tmp0_ref, tmp1_ref, tmp2_ref, tmp3_ref,
                row_ref, idx_ref, ostage_ref, shared_ref):
    tmp_refs = (tmp0_ref, tmp1_ref, tmp2_ref, tmp3_ref)
    wid = lax.axis_index("c") * 16 + lax.axis_index("s")
    b = wid // nb
    cb = wid % nb
    pchunk = p_total * _NSAMPLE
    pltpu.sync_copy(xyz_t_hbm.at[pl.ds((b * 3 + 0) * n, n)], x_ref)
    pltpu.sync_copy(xyz_t_hbm.at[pl.ds((b * 3 + 1) * n, n)], y_ref)
    pltpu.sync_copy(xyz_t_hbm.at[pl.ds((b * 3 + 2) * n, n)], z_ref)
    cstart = cb * pb
    pltpu.sync_copy(nxyz_t_hbm.at[pl.ds((b * 3 + 0) * p_total + cstart, pb)],
                    cx_ref)
    pltpu.sync_copy(nxyz_t_hbm.at[pl.ds((b * 3 + 1) * p_total + cstart, pb)],
                    cy_ref)
    pltpu.sync_copy(nxyz_t_hbm.at[pl.ds((b * 3 + 2) * p_total + cstart, pb)],
                    cz_ref)
    iota = lax.iota(jnp.int32, _L)
    zeros16 = jnp.zeros((_L,), jnp.int32)
    nblk = n // (_L * _CB)

    def per_group(g, carry):
        p0 = g * _K
        cxs, cys, czs = [], [], []
        for k in range(_K):
            pv = zeros16 + (p0 + k)
            cxs.append(plsc.load_gather(cx_ref, [pv]))
            cys.append(plsc.load_gather(cy_ref, [pv]))
            czs.append(plsc.load_gather(cz_ref, [pv]))
            tmp_refs[k][pl.ds(0, _L)] = zeros16

        def chunk(j, cnts):
            base = j * _L
            px = x_ref[pl.ds(base, _L)]
            py = y_ref[pl.ds(base, _L)]
            pz = z_ref[pl.ds(base, _L)]
            iv = iota + base
            out = []
            for k in range(_K):
                dx = cxs[k] - px
                dy = cys[k] - py
                dz = czs[k] - pz
                d2 = (dx * dx + dy * dy) + dz * dz
                m = d2 < _RADIUS2
                incl = plsc.cumsum(m.astype(jnp.int32))
                pos = jnp.minimum(jnp.maximum(cnts[k] + incl, 0), 47)
                plsc.store_scatter(tmp_refs[k], [pos], iv, mask=m)
                out.append(cnts[k] + plsc.all_reduce_population_count(m))
            return tuple(out)

        def blk_cond(state):
            blk = state[0]
            cnts = state[1:]
            cmin = jnp.minimum(jnp.minimum(cnts[0], cnts[1]),
                               jnp.minimum(cnts[2], cnts[3]))
            return jnp.logical_and(blk < nblk, jnp.min(cmin) < _NSAMPLE - 1)

        def blk_body(state):
            blk = state[0]
            cnts = plsc.parallel_loop(blk * _CB, (blk + 1) * _CB, unroll=4,
                                      carry=state[1:])(chunk)
            return (blk + 1,) + cnts

        state = (jnp.int32(0),) + tuple(zeros16 - 1 for _ in range(_K))
        state = lax.while_loop(blk_cond, blk_body, state)
        cnts_fin = state[1:]

        for k in range(_K):
            cnt = cnts_fin[k] + 1
            v0 = tmp_refs[k][pl.ds(0, _L)]
            first = zeros16 + jnp.min(
                jnp.where(iota == 0, v0, jnp.int32(2**30)))
            for h in range(_NSAMPLE // _L):
                lane = iota + (_L * h)
                v = tmp_refs[k][pl.ds(_L * h, _L)]
                ov = jnp.where(lane < cnt, v, first)
                goff = (p0 + k) * _NSAMPLE + _L * h
                stage_ref[pl.ds(goff, _L)] = ov
                gstage_ref[pl.ds(goff, _L)] = (
                    plsc.load_gather(x_ref, [ov]) - cxs[k])
                gstage_ref[pl.ds(pb * _NSAMPLE + goff, _L)] = (
                    plsc.load_gather(y_ref, [ov]) - cys[k])
                gstage_ref[pl.ds(2 * pb * _NSAMPLE + goff, _L)] = (
                    plsc.load_gather(z_ref, [ov]) - czs[k])
        return carry

    lax.fori_loop(0, pb // _K, per_group, 0)
    pltpu.sync_copy(
        stage_ref,
        shared_ref.at[pl.ds((b % 2) * pchunk + cstart * _NSAMPLE,
                            pb * _NSAMPLE)])
    for ch in range(3):
        pltpu.sync_copy(
            gstage_ref.at[pl.ds(ch * pb * _NSAMPLE, pb * _NSAMPLE)],
            out_hbm.at[pl.ds(((b * nrows + ch) * p_total + cstart) * _NSAMPLE,
                             pb * _NSAMPLE)])
    plsc.subcore_barrier()

    nfeat = nrows - 3
    pltpu.sync_copy(shared_ref.at[pl.ds((b % 2) * pchunk, pchunk)], idx_ref)
    pq = p_total // _Q

    def per_task(t, carry):
        c = cb + nb * t
        pltpu.sync_copy(feat_hbm.at[pl.ds((b * nfeat + c) * n, n)], row_ref)

        def quarter(q, carry2):
            qbase = q * pq

            @plsc.parallel_loop(0, pq, unroll=4)
            def per_p(p):
                for h in range(_NSAMPLE // _L):
                    loc = p * _NSAMPLE + _L * h
                    iv = idx_ref[pl.ds(qbase * _NSAMPLE + loc, _L)]
                    ostage_ref[pl.ds(loc, _L)] = plsc.load_gather(
                        row_ref, [iv])

            pltpu.sync_copy(
                ostage_ref,
                out_hbm.at[pl.ds(((b * nrows + 3 + c) * p_total + qbase)
                                 * _NSAMPLE, pq * _NSAMPLE)])
            return carry2

        lax.fori_loop(0, _Q, quarter, 0)
        return carry

    lax.fori_loop(0, nfeat // nb, per_task, 0)


def kernel(xyz, new_xyz, features):
    B, N, _ = xyz.shape
    P = new_xyz.shape[1]
    C = features.shape[1]
    nrows = C + 3
    nb = 32 // B
    pb = P // nb
    mesh = plsc.VectorSubcoreMesh(core_axis_name="c", subcore_axis_name="s",
                                  num_cores=2, num_subcores=16)

    xyz_t = jnp.transpose(xyz, (0, 2, 1)).reshape(-1)
    nxyz_t = jnp.transpose(new_xyz, (0, 2, 1)).reshape(-1)
    feat_flat = features.reshape(-1)

    fused = pl.kernel(
        functools.partial(_fused_body, nb, pb, N, P, nrows),
        out_type=jax.ShapeDtypeStruct((B * nrows * P * _NSAMPLE,),
                                      jnp.float32),
        mesh=mesh,
        compiler_params=pltpu.CompilerParams(needs_layout_passes=False),
        scratch_types=[
            pltpu.VMEM((N,), jnp.float32),
            pltpu.VMEM((N,), jnp.float32),
            pltpu.VMEM((N,), jnp.float32),
            pltpu.VMEM((pb,), jnp.float32),
            pltpu.VMEM((pb,), jnp.float32),
            pltpu.VMEM((pb,), jnp.float32),
            pltpu.VMEM((pb * _NSAMPLE,), jnp.int32),
            pltpu.VMEM((3 * pb * _NSAMPLE,), jnp.float32),
            pltpu.VMEM((48,), jnp.int32),
            pltpu.VMEM((48,), jnp.int32),
            pltpu.VMEM((48,), jnp.int32),
            pltpu.VMEM((48,), jnp.int32),
            pltpu.VMEM((N,), jnp.float32),
            pltpu.VMEM((P * _NSAMPLE,), jnp.int32),
            pltpu.VMEM((P * _NSAMPLE // _Q,), jnp.float32),
            pltpu.VMEM_SHARED((2 * P * _NSAMPLE,), jnp.int32),
        ],
    )
    out = fused(xyz_t, nxyz_t, feat_flat)
    return out.reshape(B, nrows, P, _NSAMPLE)
